# baseline (speedup 1.0000x reference)
import functools
import math

import jax
import jax.numpy as jnp
from jax.experimental import pallas as pl
from jax.experimental.pallas import tpu as pltpu

_ND = 50
_HD = 10
_NI = 101
_OW = 30


def _body(x_sm, hw_v, wt_v, tbl_hbm, out_v, tbl_v, sem):
    cp = pltpu.make_async_copy(tbl_hbm, tbl_v, sem)
    cp.start()
    v = x_sm[0, 0] * (_NI - 1)
    t = (v + 0.5).astype(jnp.int32)
    tie_odd = jnp.logical_and(t.astype(jnp.float32) == v + 0.5, (t & 1) == 1)
    idx = jnp.clip(jnp.where(tie_odd, t - 1, t), 0, _NI - 1)
    sim = jnp.dot(hw_v[...], wt_v[...]) * (1.0 / math.sqrt(_HD))
    e = jnp.exp(jnp.sin(sim))
    p = e / jnp.sum(e)
    pw = jnp.transpose(p)
    lane = jax.lax.broadcasted_iota(jnp.int32, (1, 1, _NI), 2)
    cp.wait()
    sel = jnp.sum(jnp.where(lane == idx, tbl_v[...], 0.0), axis=2)
    red = jnp.sum(pw * sel, axis=0, keepdims=True)
    out_v[...] = jnp.concatenate(
        [red[:, 5 * r:5 * r + 5] for r in range(6)], axis=0)


@functools.partial(jax.jit, static_argnames=())
def kernel(x, hw, hw_embed_weight, hpn_tables):
    wt = hw_embed_weight.T
    tblt = jnp.transpose(hpn_tables, (0, 2, 1))
    out = pl.pallas_call(
        _body,
        out_shape=jax.ShapeDtypeStruct((6, 5), jnp.float32),
        in_specs=[
            pl.BlockSpec(memory_space=pltpu.SMEM),
            pl.BlockSpec(memory_space=pltpu.VMEM),
            pl.BlockSpec(memory_space=pltpu.VMEM),
            pl.BlockSpec(memory_space=pl.ANY),
        ],
        out_specs=pl.BlockSpec(memory_space=pltpu.VMEM),
        scratch_shapes=[
            pltpu.VMEM((_ND, _OW, _NI), jnp.float32),
            pltpu.SemaphoreType.DMA,
        ],
    )(x, hw.reshape(1, _HD), wt, tblt)
    return out


# speedup vs baseline: 1.2284x; 1.2284x over previous
import functools
import math

import jax
import jax.numpy as jnp
from jax.experimental import pallas as pl
from jax.experimental.pallas import tpu as pltpu

_ND = 50
_HD = 10
_NI = 101
_OW = 30


def _body(x_sm, hw_v, wt_v, tbl_v, out_v):
    v = x_sm[0, 0] * (_NI - 1)
    t = (v + 0.5).astype(jnp.int32)
    tie_odd = jnp.logical_and(t.astype(jnp.float32) == v + 0.5, (t & 1) == 1)
    idx = jnp.clip(jnp.where(tie_odd, t - 1, t), 0, _NI - 1)
    sim = jnp.dot(hw_v[...], wt_v[...]) * (1.0 / math.sqrt(_HD))
    e = jnp.exp(jnp.sin(sim))
    p = e / jnp.sum(e)
    pw = jnp.transpose(p)
    lane = jax.lax.broadcasted_iota(jnp.int32, (1, 1, _NI), 2)
    sel = jnp.sum(jnp.where(lane == idx, tbl_v[...], 0.0), axis=2)
    red = jnp.sum(pw * sel, axis=0, keepdims=True)
    out_v[...] = jnp.concatenate(
        [red[:, 5 * r:5 * r + 5] for r in range(6)], axis=0)


@functools.partial(jax.jit, static_argnames=())
def kernel(x, hw, hw_embed_weight, hpn_tables):
    wt = hw_embed_weight.T
    tblt = jnp.transpose(hpn_tables, (0, 2, 1))
    out = pl.pallas_call(
        _body,
        out_shape=jax.ShapeDtypeStruct((6, 5), jnp.float32),
        in_specs=[
            pl.BlockSpec(memory_space=pltpu.SMEM),
            pl.BlockSpec(memory_space=pltpu.VMEM),
            pl.BlockSpec(memory_space=pltpu.VMEM),
            pl.BlockSpec(memory_space=pltpu.VMEM),
        ],
        out_specs=pl.BlockSpec(memory_space=pltpu.VMEM),
    )(x, hw.reshape(1, _HD), wt, tblt)
    return out
